# Initial kernel scaffold; baseline (speedup 1.0000x reference)
import jax
import jax.numpy as jnp
from jax.experimental import pallas as pl


def kernel(nodes_feature, edges, edges_feature, params):
    raise NotImplementedError("write your pallas kernel here")


# speedup vs baseline: 2.1978x; 2.1978x over previous
import functools

import jax
import jax.numpy as jnp
from jax import lax
from jax.experimental import pallas as pl
from jax.experimental.pallas import tpu as pltpu
from jax.experimental.pallas import tpu_sc as plsc

N_NODES = 10000
D_STATE = 32
NC, NS = 2, 16
NW = NC * NS
E_BLK = 128
EDGE_PAD = 327680
PER_W = EDGE_PAD // NW
NBLK = PER_W // E_BLK
N_PAD = 10016
STRIPE = N_PAD // NS

_MESH = dict(core_axis_name="c", subcore_axis_name="s")


def _sc_gather(h_tbl, src_idx, dst_idx):
    out = (jax.ShapeDtypeStruct((EDGE_PAD, D_STATE), jnp.float32),
           jax.ShapeDtypeStruct((EDGE_PAD, D_STATE), jnp.float32))

    @functools.partial(
        pl.kernel, mesh=plsc.VectorSubcoreMesh(**_MESH), out_type=out,
        compiler_params=pltpu.CompilerParams(use_tc_tiling_on_sc=False),
        scratch_types=[
            pltpu.VMEM((PER_W,), jnp.int32),
            pltpu.VMEM((PER_W,), jnp.int32),
            pltpu.VMEM((E_BLK, D_STATE), jnp.float32),
            pltpu.VMEM((E_BLK, D_STATE), jnp.float32),
            pltpu.SemaphoreType.DMA,
        ])
    def k(h_hbm, src_hbm, dst_hbm, hs_hbm, hd_hbm, idx_s, idx_d, buf_s, buf_d,
          sem):
        wid = lax.axis_index("s") * NC + lax.axis_index("c")
        base = wid * PER_W
        pltpu.sync_copy(src_hbm.at[pl.ds(base, PER_W)], idx_s)
        pltpu.sync_copy(dst_hbm.at[pl.ds(base, PER_W)], idx_d)

        @pl.loop(0, NBLK)
        def _(j):
            s = j * E_BLK
            c1 = pltpu.async_copy(h_hbm.at[idx_s.at[pl.ds(s, E_BLK)]], buf_s,
                                  sem)
            c2 = pltpu.async_copy(h_hbm.at[idx_d.at[pl.ds(s, E_BLK)]], buf_d,
                                  sem)
            c1.wait()
            c2.wait()
            pltpu.sync_copy(buf_s, hs_hbm.at[pl.ds(base + s, E_BLK)])
            pltpu.sync_copy(buf_d, hd_hbm.at[pl.ds(base + s, E_BLK)])

    return k(h_tbl, src_idx, dst_idx)


def _sc_scatter(msg, dst2d):
    @functools.partial(
        pl.kernel, mesh=plsc.VectorSubcoreMesh(**_MESH),
        out_type=jax.ShapeDtypeStruct((NC, N_PAD, D_STATE), jnp.float32),
        compiler_params=pltpu.CompilerParams(use_tc_tiling_on_sc=False),
        scratch_types=[
            pltpu.VMEM_SHARED((N_PAD, D_STATE), jnp.float32),
            pltpu.VMEM((NBLK, E_BLK), jnp.int32),
            pltpu.VMEM((E_BLK, D_STATE), jnp.float32),
            pltpu.VMEM((STRIPE, D_STATE), jnp.float32),
        ])
    def k(msg_hbm, dst_hbm, out_hbm, acc, idx, mbuf, zbuf):
        cid = lax.axis_index("c")
        sid = lax.axis_index("s")
        wid = sid * NC + cid
        z = jnp.zeros((16,), jnp.float32)

        @pl.loop(0, STRIPE)
        def _(r):
            zbuf[r, pl.ds(0, 16)] = z
            zbuf[r, pl.ds(16, 16)] = z

        pltpu.sync_copy(zbuf, acc.at[pl.ds(sid * STRIPE, STRIPE)])
        plsc.subcore_barrier()

        pltpu.sync_copy(dst_hbm.at[pl.ds(wid * NBLK, NBLK)], idx)

        @pl.loop(0, NBLK)
        def _(j):
            pltpu.sync_copy(msg_hbm.at[pl.ds(wid * PER_W + j * E_BLK, E_BLK)],
                            mbuf)
            pltpu.sync_copy(mbuf, acc.at[idx.at[j]], add=True)

        plsc.subcore_barrier()
        pltpu.sync_copy(acc.at[pl.ds(sid * STRIPE, STRIPE)],
                        out_hbm.at[cid].at[pl.ds(sid * STRIPE, STRIPE)])

    return k(msg, dst2d)


def _dot(a, b):
    return jnp.dot(a, b, preferred_element_type=jnp.float32)


def _node_mlp_body(x_ref, w1_ref, b1_ref, w2_ref, b2_ref, o_ref):
    hid = jnp.maximum(_dot(x_ref[...], w1_ref[...]) + b1_ref[...], 0.0)
    o_ref[...] = _dot(hid, w2_ref[...]) + b2_ref[...]


def _node_mlp(x, w1, b1, w2, b2, d_out):
    return pl.pallas_call(
        _node_mlp_body,
        out_shape=jax.ShapeDtypeStruct((x.shape[0], d_out), jnp.float32),
    )(x, w1, b1, w2, b2)


def _edge_body(hs_ref, hd_ref, ef_ref, w1s_ref, w1d_ref, w1e_ref, b1_ref,
               w2m_ref, b2m_ref, w2a_ref, b2a_ref, o_ref):
    u = (_dot(hs_ref[...], w1s_ref[...]) + _dot(hd_ref[...], w1d_ref[...])
         + _dot(ef_ref[...], w1e_ref[...]) + b1_ref[...])
    u = jnp.maximum(u, 0.0)
    m = _dot(u[:, :D_STATE], w2m_ref[...]) + b2m_ref[...]
    a = jax.nn.sigmoid(_dot(u[:, D_STATE:], w2a_ref[...]) + b2a_ref[...])
    o_ref[...] = m * a


_EB = 8192


def _edge_mlp(hs, hd, efp, w1s, w1d, w1e, b1, w2m, b2m, w2a, b2a):
    full = lambda shape: pl.BlockSpec(shape, lambda i: (0, 0))
    return pl.pallas_call(
        _edge_body,
        grid=(EDGE_PAD // _EB,),
        in_specs=[
            pl.BlockSpec((_EB, D_STATE), lambda i: (i, 0)),
            pl.BlockSpec((_EB, D_STATE), lambda i: (i, 0)),
            pl.BlockSpec((_EB, 16), lambda i: (i, 0)),
            full((D_STATE, 64)), full((D_STATE, 64)), full((16, 64)),
            full((1, 64)), full((D_STATE, D_STATE)), full((1, D_STATE)),
            full((D_STATE, D_STATE)), full((1, D_STATE)),
        ],
        out_specs=pl.BlockSpec((_EB, D_STATE), lambda i: (i, 0)),
        out_shape=jax.ShapeDtypeStruct((EDGE_PAD, D_STATE), jnp.float32),
        compiler_params=pltpu.CompilerParams(
            dimension_semantics=("parallel",)),
    )(hs, hd, efp, w1s, w1d, w1e, b1, w2m, b2m, w2a, b2a)


def _gru_body(p_ref, h_ref, wih_ref, bih_ref, whh_ref, bhh_ref, o_ref):
    ms = (p_ref[0] + p_ref[1])[:N_NODES]
    h = h_ref[...]
    gi = _dot(ms, wih_ref[...]) + bih_ref[...]
    gh = _dot(h, whh_ref[...]) + bhh_ref[...]
    r = jax.nn.sigmoid(gi[:, :D_STATE] + gh[:, :D_STATE])
    z = jax.nn.sigmoid(gi[:, D_STATE:2 * D_STATE] + gh[:, D_STATE:2 * D_STATE])
    n = jnp.tanh(gi[:, 2 * D_STATE:] + r * gh[:, 2 * D_STATE:])
    o_ref[...] = (1.0 - z) * n + z * h


def _gru(part, h, wih, bih, whh, bhh):
    return pl.pallas_call(
        _gru_body,
        out_shape=jax.ShapeDtypeStruct((N_NODES, D_STATE), jnp.float32),
    )(part, h, wih, bih, whh, bhh)


def _res_body(h_ref, old_ref, w_ref, b_ref, o_ref, orelu_ref):
    x = (_dot(h_ref[...], w_ref[:D_STATE]) + _dot(old_ref[...], w_ref[D_STATE:])
         + b_ref[...])
    o_ref[...] = x
    orelu_ref[...] = jnp.maximum(x, 0.0)


def _res(h, old, w, b):
    return pl.pallas_call(
        _res_body,
        out_shape=(jax.ShapeDtypeStruct((N_NODES, D_STATE), jnp.float32),
                   jax.ShapeDtypeStruct((N_NODES, D_STATE), jnp.float32)),
    )(h, old, w, b)


def kernel(nodes_feature, edges, edges_feature, params):
    p = params
    n_edges = edges.shape[0]
    npad = EDGE_PAD - n_edges
    src = edges[:, 0].astype(jnp.int32)
    dst = edges[:, 1].astype(jnp.int32)
    srcp = jnp.pad(src, (0, npad))
    dstp = jnp.pad(dst, (0, npad))
    dst_sc = jnp.pad(dst, (0, npad), constant_values=N_NODES)
    dst2d = dst_sc.reshape(EDGE_PAD // E_BLK, E_BLK)
    efp = jnp.pad(edges_feature, ((0, npad), (0, 0)))

    h = _node_mlp(nodes_feature, p['in_W1'].T, p['in_b1'][None],
                  p['in_W2'].T, p['in_b2'][None], D_STATE)

    h_relu = None
    for i in range(2):
        old = h
        if i > 0:
            h = h_relu
        mW1, aW1 = p['msg_W1_%d' % i], p['att_W1_%d' % i]
        w1s = jnp.concatenate([mW1[:, :32], aW1[:, :32]], 0).T
        w1d = jnp.concatenate([mW1[:, 32:64], aW1[:, 32:64]], 0).T
        w1e = jnp.concatenate([mW1[:, 64:], aW1[:, 64:]], 0).T
        b1 = jnp.concatenate([p['msg_b1_%d' % i], p['att_b1_%d' % i]])[None]
        w2m, b2m = p['msg_W2_%d' % i].T, p['msg_b2_%d' % i][None]
        w2a, b2a = p['att_W2_%d' % i].T, p['att_b2_%d' % i][None]
        wih, bih = p['gru_Wih_%d' % i].T, p['gru_bih_%d' % i][None]
        whh, bhh = p['gru_Whh_%d' % i].T, p['gru_bhh_%d' % i][None]
        for _ in range(2):
            hs, hd = _sc_gather(h, srcp, dstp)
            msg = _edge_mlp(hs, hd, efp, w1s, w1d, w1e, b1, w2m, b2m, w2a, b2a)
            part = _sc_scatter(msg, dst2d)
            h = _gru(part, h, wih, bih, whh, bhh)
        h, h_relu = _res(h, old, p['res_W_%d' % i].T, p['res_b_%d' % i][None])

    return _node_mlp(h, p['ro_W1'].T, p['ro_b1'][None],
                     p['ro_W2'].T, p['ro_b2'][None], 64)


# speedup vs baseline: 2.4020x; 1.0929x over previous
import functools

import jax
import jax.numpy as jnp
from jax import lax
from jax.experimental import pallas as pl
from jax.experimental.pallas import tpu as pltpu
from jax.experimental.pallas import tpu_sc as plsc

N_NODES = 10000
D_STATE = 32
NC, NS = 2, 16
NW = NC * NS
E_BLK = 128
EDGE_PAD = 327680
PER_W = EDGE_PAD // NW
NBLK = PER_W // E_BLK
N_PAD = 10016
STRIPE = N_PAD // NS

_MESH = dict(core_axis_name="c", subcore_axis_name="s")
_GK = 4


def _sc_gather(h_tbl, src_idx, dst_idx):
    out = (jax.ShapeDtypeStruct((EDGE_PAD, D_STATE), jnp.float32),
           jax.ShapeDtypeStruct((EDGE_PAD, D_STATE), jnp.float32))

    @functools.partial(
        pl.kernel, mesh=plsc.VectorSubcoreMesh(**_MESH), out_type=out,
        compiler_params=pltpu.CompilerParams(use_tc_tiling_on_sc=False),
        scratch_types=[
            pltpu.VMEM((PER_W,), jnp.int32),
            pltpu.VMEM((PER_W,), jnp.int32),
            pltpu.VMEM((_GK, E_BLK, D_STATE), jnp.float32),
            pltpu.VMEM((_GK, E_BLK, D_STATE), jnp.float32),
            pltpu.SemaphoreType.DMA,
            pltpu.SemaphoreType.DMA,
        ])
    def k(h_hbm, src_hbm, dst_hbm, hs_hbm, hd_hbm, idx_s, idx_d, buf_s, buf_d,
          gsem, wsem):
        wid = lax.axis_index("s") * NC + lax.axis_index("c")
        base = wid * PER_W
        pltpu.sync_copy(src_hbm.at[pl.ds(base, PER_W)], idx_s)
        pltpu.sync_copy(dst_hbm.at[pl.ds(base, PER_W)], idx_d)

        @pl.loop(0, NBLK, step=_GK)
        def _(j0):
            s0 = j0 * E_BLK
            gathers = []
            for b in range(_GK):
                s = s0 + b * E_BLK
                gathers.append(pltpu.async_copy(
                    h_hbm.at[idx_s.at[pl.ds(s, E_BLK)]], buf_s.at[b], gsem))
                gathers.append(pltpu.async_copy(
                    h_hbm.at[idx_d.at[pl.ds(s, E_BLK)]], buf_d.at[b], gsem))
            writes = []
            for b in range(_GK):
                s = s0 + b * E_BLK
                gathers[2 * b].wait()
                writes.append(pltpu.async_copy(
                    buf_s.at[b], hs_hbm.at[pl.ds(base + s, E_BLK)], wsem))
                gathers[2 * b + 1].wait()
                writes.append(pltpu.async_copy(
                    buf_d.at[b], hd_hbm.at[pl.ds(base + s, E_BLK)], wsem))
            for w in writes:
                w.wait()

    return k(h_tbl, src_idx, dst_idx)


def _sc_scatter(msg, dst2d):
    @functools.partial(
        pl.kernel, mesh=plsc.VectorSubcoreMesh(**_MESH),
        out_type=jax.ShapeDtypeStruct((NC, N_PAD, D_STATE), jnp.float32),
        compiler_params=pltpu.CompilerParams(use_tc_tiling_on_sc=False),
        scratch_types=[
            pltpu.VMEM_SHARED((N_PAD, D_STATE), jnp.float32),
            pltpu.VMEM((NBLK, E_BLK), jnp.int32),
            pltpu.VMEM((_GK, E_BLK, D_STATE), jnp.float32),
            pltpu.VMEM((STRIPE, D_STATE), jnp.float32),
            pltpu.SemaphoreType.DMA,
        ])
    def k(msg_hbm, dst_hbm, out_hbm, acc, idx, mbuf, zbuf, lsem):
        cid = lax.axis_index("c")
        sid = lax.axis_index("s")
        wid = sid * NC + cid
        z = jnp.zeros((16,), jnp.float32)

        @pl.loop(0, STRIPE)
        def _(r):
            zbuf[r, pl.ds(0, 16)] = z
            zbuf[r, pl.ds(16, 16)] = z

        pltpu.sync_copy(zbuf, acc.at[pl.ds(sid * STRIPE, STRIPE)])
        plsc.subcore_barrier()

        pltpu.sync_copy(dst_hbm.at[pl.ds(wid * NBLK, NBLK)], idx)

        @pl.loop(0, NBLK, step=_GK)
        def _(j0):
            loads = []
            for b in range(_GK):
                loads.append(pltpu.async_copy(
                    msg_hbm.at[pl.ds(wid * PER_W + (j0 + b) * E_BLK, E_BLK)],
                    mbuf.at[b], lsem))
            for b in range(_GK):
                loads[b].wait()
                pltpu.sync_copy(mbuf.at[b], acc.at[idx.at[j0 + b]], add=True)

        plsc.subcore_barrier()
        pltpu.sync_copy(acc.at[pl.ds(sid * STRIPE, STRIPE)],
                        out_hbm.at[cid].at[pl.ds(sid * STRIPE, STRIPE)])

    return k(msg, dst2d)


def _dot(a, b):
    return jnp.dot(a, b, preferred_element_type=jnp.float32)


def _node_mlp_body(x_ref, w1_ref, b1_ref, w2_ref, b2_ref, o_ref):
    hid = jnp.maximum(_dot(x_ref[...], w1_ref[...]) + b1_ref[...], 0.0)
    o_ref[...] = _dot(hid, w2_ref[...]) + b2_ref[...]


def _node_mlp(x, w1, b1, w2, b2, d_out):
    return pl.pallas_call(
        _node_mlp_body,
        out_shape=jax.ShapeDtypeStruct((x.shape[0], d_out), jnp.float32),
    )(x, w1, b1, w2, b2)


def _edge_body(hs_ref, hd_ref, ef_ref, w1s_ref, w1d_ref, w1e_ref, b1_ref,
               w2m_ref, b2m_ref, w2a_ref, b2a_ref, o_ref):
    u = (_dot(hs_ref[...], w1s_ref[...]) + _dot(hd_ref[...], w1d_ref[...])
         + _dot(ef_ref[...], w1e_ref[...]) + b1_ref[...])
    u = jnp.maximum(u, 0.0)
    m = _dot(u[:, :D_STATE], w2m_ref[...]) + b2m_ref[...]
    a = jax.nn.sigmoid(_dot(u[:, D_STATE:], w2a_ref[...]) + b2a_ref[...])
    o_ref[...] = m * a


_EB = 8192


def _edge_mlp(hs, hd, efp, w1s, w1d, w1e, b1, w2m, b2m, w2a, b2a):
    full = lambda shape: pl.BlockSpec(shape, lambda i: (0, 0))
    return pl.pallas_call(
        _edge_body,
        grid=(EDGE_PAD // _EB,),
        in_specs=[
            pl.BlockSpec((_EB, D_STATE), lambda i: (i, 0)),
            pl.BlockSpec((_EB, D_STATE), lambda i: (i, 0)),
            pl.BlockSpec((_EB, 16), lambda i: (i, 0)),
            full((D_STATE, 64)), full((D_STATE, 64)), full((16, 64)),
            full((1, 64)), full((D_STATE, D_STATE)), full((1, D_STATE)),
            full((D_STATE, D_STATE)), full((1, D_STATE)),
        ],
        out_specs=pl.BlockSpec((_EB, D_STATE), lambda i: (i, 0)),
        out_shape=jax.ShapeDtypeStruct((EDGE_PAD, D_STATE), jnp.float32),
        compiler_params=pltpu.CompilerParams(
            dimension_semantics=("parallel",)),
    )(hs, hd, efp, w1s, w1d, w1e, b1, w2m, b2m, w2a, b2a)


def _gru_body(p_ref, h_ref, wih_ref, bih_ref, whh_ref, bhh_ref, o_ref):
    ms = (p_ref[0] + p_ref[1])[:N_NODES]
    h = h_ref[...]
    gi = _dot(ms, wih_ref[...]) + bih_ref[...]
    gh = _dot(h, whh_ref[...]) + bhh_ref[...]
    r = jax.nn.sigmoid(gi[:, :D_STATE] + gh[:, :D_STATE])
    z = jax.nn.sigmoid(gi[:, D_STATE:2 * D_STATE] + gh[:, D_STATE:2 * D_STATE])
    n = jnp.tanh(gi[:, 2 * D_STATE:] + r * gh[:, 2 * D_STATE:])
    o_ref[...] = (1.0 - z) * n + z * h


def _gru(part, h, wih, bih, whh, bhh):
    return pl.pallas_call(
        _gru_body,
        out_shape=jax.ShapeDtypeStruct((N_NODES, D_STATE), jnp.float32),
    )(part, h, wih, bih, whh, bhh)


def _res_body(h_ref, old_ref, w_ref, b_ref, o_ref, orelu_ref):
    x = (_dot(h_ref[...], w_ref[:D_STATE]) + _dot(old_ref[...], w_ref[D_STATE:])
         + b_ref[...])
    o_ref[...] = x
    orelu_ref[...] = jnp.maximum(x, 0.0)


def _res(h, old, w, b):
    return pl.pallas_call(
        _res_body,
        out_shape=(jax.ShapeDtypeStruct((N_NODES, D_STATE), jnp.float32),
                   jax.ShapeDtypeStruct((N_NODES, D_STATE), jnp.float32)),
    )(h, old, w, b)


def kernel(nodes_feature, edges, edges_feature, params):
    p = params
    n_edges = edges.shape[0]
    npad = EDGE_PAD - n_edges
    src = edges[:, 0].astype(jnp.int32)
    dst = edges[:, 1].astype(jnp.int32)
    srcp = jnp.pad(src, (0, npad))
    dstp = jnp.pad(dst, (0, npad))
    dst_sc = jnp.pad(dst, (0, npad), constant_values=N_NODES)
    dst2d = dst_sc.reshape(EDGE_PAD // E_BLK, E_BLK)
    efp = jnp.pad(edges_feature, ((0, npad), (0, 0)))

    h = _node_mlp(nodes_feature, p['in_W1'].T, p['in_b1'][None],
                  p['in_W2'].T, p['in_b2'][None], D_STATE)

    h_relu = None
    for i in range(2):
        old = h
        if i > 0:
            h = h_relu
        mW1, aW1 = p['msg_W1_%d' % i], p['att_W1_%d' % i]
        w1s = jnp.concatenate([mW1[:, :32], aW1[:, :32]], 0).T
        w1d = jnp.concatenate([mW1[:, 32:64], aW1[:, 32:64]], 0).T
        w1e = jnp.concatenate([mW1[:, 64:], aW1[:, 64:]], 0).T
        b1 = jnp.concatenate([p['msg_b1_%d' % i], p['att_b1_%d' % i]])[None]
        w2m, b2m = p['msg_W2_%d' % i].T, p['msg_b2_%d' % i][None]
        w2a, b2a = p['att_W2_%d' % i].T, p['att_b2_%d' % i][None]
        wih, bih = p['gru_Wih_%d' % i].T, p['gru_bih_%d' % i][None]
        whh, bhh = p['gru_Whh_%d' % i].T, p['gru_bhh_%d' % i][None]
        for _ in range(2):
            hs, hd = _sc_gather(h, srcp, dstp)
            msg = _edge_mlp(hs, hd, efp, w1s, w1d, w1e, b1, w2m, b2m, w2a, b2a)
            part = _sc_scatter(msg, dst2d)
            h = _gru(part, h, wih, bih, whh, bhh)
        h, h_relu = _res(h, old, p['res_W_%d' % i].T, p['res_b_%d' % i][None])

    return _node_mlp(h, p['ro_W1'].T, p['ro_b1'][None],
                     p['ro_W2'].T, p['ro_b2'][None], 64)


# speedup vs baseline: 3.1597x; 1.3155x over previous
import functools

import jax
import jax.numpy as jnp
from jax import lax
from jax.experimental import pallas as pl
from jax.experimental.pallas import tpu as pltpu
from jax.experimental.pallas import tpu_sc as plsc

N_NODES = 10000
D_STATE = 32
NC, NS = 2, 16
NW = NC * NS
E_BLK = 128
EDGE_PAD = 327680
PER_W = EDGE_PAD // NW
NBLK = PER_W // E_BLK
N_PAD = 10016
STRIPE = N_PAD // NS

_MESH = dict(core_axis_name="c", subcore_axis_name="s")
_GK = 4


def _sc_gather(h_tbl, src_idx, dst_idx):
    out = (jax.ShapeDtypeStruct((EDGE_PAD, D_STATE), jnp.float32),
           jax.ShapeDtypeStruct((EDGE_PAD, D_STATE), jnp.float32))

    @functools.partial(
        pl.kernel, mesh=plsc.VectorSubcoreMesh(**_MESH), out_type=out,
        compiler_params=pltpu.CompilerParams(use_tc_tiling_on_sc=False),
        scratch_types=[
            pltpu.VMEM_SHARED((N_NODES, D_STATE), jnp.float32),
            pltpu.VMEM((PER_W,), jnp.int32),
            pltpu.VMEM((PER_W,), jnp.int32),
            pltpu.VMEM((_GK, E_BLK, D_STATE), jnp.float32),
            pltpu.VMEM((_GK, E_BLK, D_STATE), jnp.float32),
            pltpu.SemaphoreType.DMA,
            pltpu.SemaphoreType.DMA,
        ])
    def k(h_hbm, src_hbm, dst_hbm, hs_hbm, hd_hbm, h_sh, idx_s, idx_d, buf_s,
          buf_d, gsem, wsem):
        sid = lax.axis_index("s")
        wid = sid * NC + lax.axis_index("c")
        base = wid * PER_W
        pltpu.sync_copy(h_hbm.at[pl.ds(sid * (N_NODES // NS), N_NODES // NS)],
                        h_sh.at[pl.ds(sid * (N_NODES // NS), N_NODES // NS)])
        pltpu.sync_copy(src_hbm.at[pl.ds(base, PER_W)], idx_s)
        pltpu.sync_copy(dst_hbm.at[pl.ds(base, PER_W)], idx_d)
        plsc.subcore_barrier()

        @pl.loop(0, NBLK, step=_GK)
        def _(j0):
            s0 = j0 * E_BLK
            gathers = []
            for b in range(_GK):
                s = s0 + b * E_BLK
                gathers.append(pltpu.async_copy(
                    h_sh.at[idx_s.at[pl.ds(s, E_BLK)]], buf_s.at[b], gsem))
                gathers.append(pltpu.async_copy(
                    h_sh.at[idx_d.at[pl.ds(s, E_BLK)]], buf_d.at[b], gsem))
            writes = []
            for b in range(_GK):
                s = s0 + b * E_BLK
                gathers[2 * b].wait()
                writes.append(pltpu.async_copy(
                    buf_s.at[b], hs_hbm.at[pl.ds(base + s, E_BLK)], wsem))
                gathers[2 * b + 1].wait()
                writes.append(pltpu.async_copy(
                    buf_d.at[b], hd_hbm.at[pl.ds(base + s, E_BLK)], wsem))
            for w in writes:
                w.wait()

    return k(h_tbl, src_idx, dst_idx)


def _sc_scatter(msg, dst2d):
    @functools.partial(
        pl.kernel, mesh=plsc.VectorSubcoreMesh(**_MESH),
        out_type=jax.ShapeDtypeStruct((NC, N_PAD, D_STATE), jnp.float32),
        compiler_params=pltpu.CompilerParams(use_tc_tiling_on_sc=False),
        scratch_types=[
            pltpu.VMEM_SHARED((N_PAD, D_STATE), jnp.float32),
            pltpu.VMEM((NBLK, E_BLK), jnp.int32),
            pltpu.VMEM((_GK, E_BLK, D_STATE), jnp.float32),
            pltpu.VMEM((STRIPE, D_STATE), jnp.float32),
            pltpu.SemaphoreType.DMA,
        ])
    def k(msg_hbm, dst_hbm, out_hbm, acc, idx, mbuf, zbuf, lsem):
        cid = lax.axis_index("c")
        sid = lax.axis_index("s")
        wid = sid * NC + cid
        z = jnp.zeros((16,), jnp.float32)

        @pl.loop(0, STRIPE)
        def _(r):
            zbuf[r, pl.ds(0, 16)] = z
            zbuf[r, pl.ds(16, 16)] = z

        pltpu.sync_copy(zbuf, acc.at[pl.ds(sid * STRIPE, STRIPE)])
        plsc.subcore_barrier()

        pltpu.sync_copy(dst_hbm.at[pl.ds(wid * NBLK, NBLK)], idx)

        @pl.loop(0, NBLK, step=_GK)
        def _(j0):
            loads = []
            for b in range(_GK):
                loads.append(pltpu.async_copy(
                    msg_hbm.at[pl.ds(wid * PER_W + (j0 + b) * E_BLK, E_BLK)],
                    mbuf.at[b], lsem))
            for b in range(_GK):
                loads[b].wait()
                pltpu.sync_copy(mbuf.at[b], acc.at[idx.at[j0 + b]], add=True)

        plsc.subcore_barrier()
        pltpu.sync_copy(acc.at[pl.ds(sid * STRIPE, STRIPE)],
                        out_hbm.at[cid].at[pl.ds(sid * STRIPE, STRIPE)])

    return k(msg, dst2d)


def _dot(a, b):
    return jnp.dot(a, b, preferred_element_type=jnp.float32)


def _node_mlp_body(x_ref, w1_ref, b1_ref, w2_ref, b2_ref, o_ref):
    hid = jnp.maximum(_dot(x_ref[...], w1_ref[...]) + b1_ref[...], 0.0)
    o_ref[...] = _dot(hid, w2_ref[...]) + b2_ref[...]


def _node_mlp(x, w1, b1, w2, b2, d_out):
    return pl.pallas_call(
        _node_mlp_body,
        out_shape=jax.ShapeDtypeStruct((x.shape[0], d_out), jnp.float32),
    )(x, w1, b1, w2, b2)


def _edge_body(hs_ref, hd_ref, ef_ref, w1s_ref, w1d_ref, w1e_ref, b1_ref,
               w2m_ref, b2m_ref, w2a_ref, b2a_ref, o_ref):
    u = (_dot(hs_ref[...], w1s_ref[...]) + _dot(hd_ref[...], w1d_ref[...])
         + _dot(ef_ref[...], w1e_ref[...]) + b1_ref[...])
    u = jnp.maximum(u, 0.0)
    m = _dot(u[:, :D_STATE], w2m_ref[...]) + b2m_ref[...]
    a = jax.nn.sigmoid(_dot(u[:, D_STATE:], w2a_ref[...]) + b2a_ref[...])
    o_ref[...] = m * a


_EB = 8192


def _edge_mlp(hs, hd, efp, w1s, w1d, w1e, b1, w2m, b2m, w2a, b2a):
    full = lambda shape: pl.BlockSpec(shape, lambda i: (0, 0))
    return pl.pallas_call(
        _edge_body,
        grid=(EDGE_PAD // _EB,),
        in_specs=[
            pl.BlockSpec((_EB, D_STATE), lambda i: (i, 0)),
            pl.BlockSpec((_EB, D_STATE), lambda i: (i, 0)),
            pl.BlockSpec((_EB, 16), lambda i: (i, 0)),
            full((D_STATE, 64)), full((D_STATE, 64)), full((16, 64)),
            full((1, 64)), full((D_STATE, D_STATE)), full((1, D_STATE)),
            full((D_STATE, D_STATE)), full((1, D_STATE)),
        ],
        out_specs=pl.BlockSpec((_EB, D_STATE), lambda i: (i, 0)),
        out_shape=jax.ShapeDtypeStruct((EDGE_PAD, D_STATE), jnp.float32),
        compiler_params=pltpu.CompilerParams(
            dimension_semantics=("parallel",)),
    )(hs, hd, efp, w1s, w1d, w1e, b1, w2m, b2m, w2a, b2a)


def _gru_body(p_ref, h_ref, wih_ref, bih_ref, whh_ref, bhh_ref, o_ref):
    ms = (p_ref[0] + p_ref[1])[:N_NODES]
    h = h_ref[...]
    gi = _dot(ms, wih_ref[...]) + bih_ref[...]
    gh = _dot(h, whh_ref[...]) + bhh_ref[...]
    r = jax.nn.sigmoid(gi[:, :D_STATE] + gh[:, :D_STATE])
    z = jax.nn.sigmoid(gi[:, D_STATE:2 * D_STATE] + gh[:, D_STATE:2 * D_STATE])
    n = jnp.tanh(gi[:, 2 * D_STATE:] + r * gh[:, 2 * D_STATE:])
    o_ref[...] = (1.0 - z) * n + z * h


def _gru(part, h, wih, bih, whh, bhh):
    return pl.pallas_call(
        _gru_body,
        out_shape=jax.ShapeDtypeStruct((N_NODES, D_STATE), jnp.float32),
    )(part, h, wih, bih, whh, bhh)


def _res_body(h_ref, old_ref, w_ref, b_ref, o_ref, orelu_ref):
    x = (_dot(h_ref[...], w_ref[:D_STATE]) + _dot(old_ref[...], w_ref[D_STATE:])
         + b_ref[...])
    o_ref[...] = x
    orelu_ref[...] = jnp.maximum(x, 0.0)


def _res(h, old, w, b):
    return pl.pallas_call(
        _res_body,
        out_shape=(jax.ShapeDtypeStruct((N_NODES, D_STATE), jnp.float32),
                   jax.ShapeDtypeStruct((N_NODES, D_STATE), jnp.float32)),
    )(h, old, w, b)


def kernel(nodes_feature, edges, edges_feature, params):
    p = params
    n_edges = edges.shape[0]
    npad = EDGE_PAD - n_edges
    src = edges[:, 0].astype(jnp.int32)
    dst = edges[:, 1].astype(jnp.int32)
    srcp = jnp.pad(src, (0, npad))
    dstp = jnp.pad(dst, (0, npad))
    dst_sc = jnp.pad(dst, (0, npad), constant_values=N_NODES)
    dst2d = dst_sc.reshape(EDGE_PAD // E_BLK, E_BLK)
    efp = jnp.pad(edges_feature, ((0, npad), (0, 0)))

    h = _node_mlp(nodes_feature, p['in_W1'].T, p['in_b1'][None],
                  p['in_W2'].T, p['in_b2'][None], D_STATE)

    h_relu = None
    for i in range(2):
        old = h
        if i > 0:
            h = h_relu
        mW1, aW1 = p['msg_W1_%d' % i], p['att_W1_%d' % i]
        w1s = jnp.concatenate([mW1[:, :32], aW1[:, :32]], 0).T
        w1d = jnp.concatenate([mW1[:, 32:64], aW1[:, 32:64]], 0).T
        w1e = jnp.concatenate([mW1[:, 64:], aW1[:, 64:]], 0).T
        b1 = jnp.concatenate([p['msg_b1_%d' % i], p['att_b1_%d' % i]])[None]
        w2m, b2m = p['msg_W2_%d' % i].T, p['msg_b2_%d' % i][None]
        w2a, b2a = p['att_W2_%d' % i].T, p['att_b2_%d' % i][None]
        wih, bih = p['gru_Wih_%d' % i].T, p['gru_bih_%d' % i][None]
        whh, bhh = p['gru_Whh_%d' % i].T, p['gru_bhh_%d' % i][None]
        for _ in range(2):
            hs, hd = _sc_gather(h, srcp, dstp)
            msg = _edge_mlp(hs, hd, efp, w1s, w1d, w1e, b1, w2m, b2m, w2a, b2a)
            part = _sc_scatter(msg, dst2d)
            h = _gru(part, h, wih, bih, whh, bhh)
        h, h_relu = _res(h, old, p['res_W_%d' % i].T, p['res_b_%d' % i][None])

    return _node_mlp(h, p['ro_W1'].T, p['ro_b1'][None],
                     p['ro_W2'].T, p['ro_b2'][None], 64)


# speedup vs baseline: 7.4928x; 2.3714x over previous
import functools

import jax
import jax.numpy as jnp
from jax import lax
from jax.experimental import pallas as pl
from jax.experimental.pallas import tpu as pltpu
from jax.experimental.pallas import tpu_sc as plsc

N_NODES = 10000
D_STATE = 32
NC, NS = 2, 16
NW = NC * NS
E_BLK = 128
EDGE_PAD = 327680
PER_W = EDGE_PAD // NW
NBLK = PER_W // E_BLK
N_PAD = 10016
STRIPE = N_PAD // NS

_MESH = dict(core_axis_name="c", subcore_axis_name="s")
_GK = 4


def _sc_gather(h_tbl, src_idx, dst_idx):
    out = (jax.ShapeDtypeStruct((EDGE_PAD, D_STATE), jnp.float32),
           jax.ShapeDtypeStruct((EDGE_PAD, D_STATE), jnp.float32))

    @functools.partial(
        pl.kernel, mesh=plsc.VectorSubcoreMesh(**_MESH), out_type=out,
        compiler_params=pltpu.CompilerParams(use_tc_tiling_on_sc=False),
        scratch_types=[
            pltpu.VMEM_SHARED((N_NODES, D_STATE), jnp.float32),
            pltpu.VMEM((PER_W,), jnp.int32),
            pltpu.VMEM((PER_W,), jnp.int32),
            pltpu.VMEM((_GK, E_BLK, D_STATE), jnp.float32),
            pltpu.VMEM((_GK, E_BLK, D_STATE), jnp.float32),
            pltpu.SemaphoreType.DMA,
            pltpu.SemaphoreType.DMA,
        ])
    def k(h_hbm, src_hbm, dst_hbm, hs_hbm, hd_hbm, h_sh, idx_s, idx_d, buf_s,
          buf_d, gsem, wsem):
        sid = lax.axis_index("s")
        wid = sid * NC + lax.axis_index("c")
        base = wid * PER_W
        pltpu.sync_copy(h_hbm.at[pl.ds(sid * (N_NODES // NS), N_NODES // NS)],
                        h_sh.at[pl.ds(sid * (N_NODES // NS), N_NODES // NS)])
        pltpu.sync_copy(src_hbm.at[pl.ds(base, PER_W)], idx_s)
        pltpu.sync_copy(dst_hbm.at[pl.ds(base, PER_W)], idx_d)
        plsc.subcore_barrier()

        @pl.loop(0, NBLK, step=_GK)
        def _(j0):
            s0 = j0 * E_BLK
            gathers = []
            for b in range(_GK):
                s = s0 + b * E_BLK
                gathers.append(pltpu.async_copy(
                    h_sh.at[idx_s.at[pl.ds(s, E_BLK)]], buf_s.at[b], gsem))
                gathers.append(pltpu.async_copy(
                    h_sh.at[idx_d.at[pl.ds(s, E_BLK)]], buf_d.at[b], gsem))
            writes = []
            for b in range(_GK):
                s = s0 + b * E_BLK
                gathers[2 * b].wait()
                writes.append(pltpu.async_copy(
                    buf_s.at[b], hs_hbm.at[pl.ds(base + s, E_BLK)], wsem))
                gathers[2 * b + 1].wait()
                writes.append(pltpu.async_copy(
                    buf_d.at[b], hd_hbm.at[pl.ds(base + s, E_BLK)], wsem))
            for w in writes:
                w.wait()

    return k(h_tbl, src_idx, dst_idx)


def _sc_scatter(msg, dst2d):
    @functools.partial(
        pl.kernel, mesh=plsc.VectorSubcoreMesh(**_MESH),
        out_type=jax.ShapeDtypeStruct((NC, N_PAD, D_STATE), jnp.float32),
        compiler_params=pltpu.CompilerParams(use_tc_tiling_on_sc=False),
        scratch_types=[
            pltpu.VMEM_SHARED((N_PAD, D_STATE), jnp.float32),
            pltpu.VMEM((NBLK, E_BLK), jnp.int32),
            pltpu.VMEM((_GK, E_BLK, D_STATE), jnp.float32),
            pltpu.VMEM((STRIPE, D_STATE), jnp.float32),
            pltpu.SemaphoreType.DMA,
        ])
    def k(msg_hbm, dst_hbm, out_hbm, acc, idx, mbuf, zbuf, lsem):
        cid = lax.axis_index("c")
        sid = lax.axis_index("s")
        wid = sid * NC + cid
        z = jnp.zeros((16,), jnp.float32)

        @pl.loop(0, STRIPE)
        def _(r):
            zbuf[r, pl.ds(0, 16)] = z
            zbuf[r, pl.ds(16, 16)] = z

        pltpu.sync_copy(zbuf, acc.at[pl.ds(sid * STRIPE, STRIPE)])
        plsc.subcore_barrier()

        pltpu.sync_copy(dst_hbm.at[pl.ds(wid * NBLK, NBLK)], idx)

        @pl.loop(0, NBLK, step=_GK)
        def _(j0):
            loads = []
            for b in range(_GK):
                loads.append(pltpu.async_copy(
                    msg_hbm.at[pl.ds(wid * PER_W + (j0 + b) * E_BLK, E_BLK)],
                    mbuf.at[b], lsem))
            for b in range(_GK):
                loads[b].wait()
                pltpu.sync_copy(mbuf.at[b], acc.at[idx.at[j0 + b]], add=True)

        plsc.subcore_barrier()
        pltpu.sync_copy(acc.at[pl.ds(sid * STRIPE, STRIPE)],
                        out_hbm.at[cid].at[pl.ds(sid * STRIPE, STRIPE)])

    return k(msg, dst2d)


def _dot(a, b):
    return jnp.dot(a, b, preferred_element_type=jnp.float32)


def _node_mlp_body(x_ref, w1_ref, b1_ref, w2_ref, b2_ref, o_ref):
    hid = jnp.maximum(_dot(x_ref[...], w1_ref[...]) + b1_ref[...], 0.0)
    o_ref[...] = _dot(hid, w2_ref[...]) + b2_ref[...]


def _node_mlp(x, w1, b1, w2, b2, d_out):
    return pl.pallas_call(
        _node_mlp_body,
        out_shape=jax.ShapeDtypeStruct((x.shape[0], d_out), jnp.float32),
    )(x, w1, b1, w2, b2)


def _edge_body(hs_ref, hd_ref, ef_ref, w1s_ref, w1d_ref, w1e_ref, b1_ref,
               w2m_ref, b2m_ref, w2a_ref, b2a_ref, o_ref):
    u = (_dot(hs_ref[...], w1s_ref[...]) + _dot(hd_ref[...], w1d_ref[...])
         + _dot(ef_ref[...], w1e_ref[...]) + b1_ref[...])
    u = jnp.maximum(u, 0.0)
    m = _dot(u, w2m_ref[...]) + b2m_ref[...]
    a = jax.nn.sigmoid(_dot(u, w2a_ref[...]) + b2a_ref[...])
    o_ref[...] = m * a


_EB = 2048


def _edge_mlp(hs, hd, ef4, w1s, w1d, w1e, b1, w2m, b2m, w2a, b2a):
    full = lambda shape: pl.BlockSpec(shape, lambda i: (0, 0))
    ep4 = EDGE_PAD // 4
    return pl.pallas_call(
        _edge_body,
        grid=(ep4 // _EB,),
        in_specs=[
            pl.BlockSpec((_EB, 128), lambda i: (i, 0)),
            pl.BlockSpec((_EB, 128), lambda i: (i, 0)),
            pl.BlockSpec((_EB, 64), lambda i: (i, 0)),
            full((128, 256)), full((128, 256)), full((64, 256)),
            full((1, 256)), full((256, 128)), full((1, 128)),
            full((256, 128)), full((1, 128)),
        ],
        out_specs=pl.BlockSpec((_EB, 128), lambda i: (i, 0)),
        out_shape=jax.ShapeDtypeStruct((ep4, 128), jnp.float32),
        compiler_params=pltpu.CompilerParams(
            dimension_semantics=("parallel",)),
    )(hs, hd, ef4, w1s, w1d, w1e, b1, w2m, b2m, w2a, b2a)


def _gru_body(p_ref, h_ref, wih_ref, bih_ref, whh_ref, bhh_ref, o_ref):
    ms = (p_ref[0] + p_ref[1])[:N_NODES]
    h = h_ref[...]
    gi = _dot(ms, wih_ref[...]) + bih_ref[...]
    gh = _dot(h, whh_ref[...]) + bhh_ref[...]
    r = jax.nn.sigmoid(gi[:, :D_STATE] + gh[:, :D_STATE])
    z = jax.nn.sigmoid(gi[:, D_STATE:2 * D_STATE] + gh[:, D_STATE:2 * D_STATE])
    n = jnp.tanh(gi[:, 2 * D_STATE:] + r * gh[:, 2 * D_STATE:])
    o_ref[...] = (1.0 - z) * n + z * h


def _gru(part, h, wih, bih, whh, bhh):
    return pl.pallas_call(
        _gru_body,
        out_shape=jax.ShapeDtypeStruct((N_NODES, D_STATE), jnp.float32),
    )(part, h, wih, bih, whh, bhh)


def _res_body(h_ref, old_ref, w_ref, b_ref, o_ref, orelu_ref):
    x = (_dot(h_ref[...], w_ref[:D_STATE]) + _dot(old_ref[...], w_ref[D_STATE:])
         + b_ref[...])
    o_ref[...] = x
    orelu_ref[...] = jnp.maximum(x, 0.0)


def _res(h, old, w, b):
    return pl.pallas_call(
        _res_body,
        out_shape=(jax.ShapeDtypeStruct((N_NODES, D_STATE), jnp.float32),
                   jax.ShapeDtypeStruct((N_NODES, D_STATE), jnp.float32)),
    )(h, old, w, b)


def kernel(nodes_feature, edges, edges_feature, params):
    p = params
    n_edges = edges.shape[0]
    npad = EDGE_PAD - n_edges
    src = edges[:, 0].astype(jnp.int32)
    dst = edges[:, 1].astype(jnp.int32)
    srcp = jnp.pad(src, (0, npad))
    dstp = jnp.pad(dst, (0, npad))
    dst_sc = jnp.pad(dst, (0, npad), constant_values=N_NODES)
    dst2d = dst_sc.reshape(EDGE_PAD // E_BLK, E_BLK)
    ef4 = jnp.pad(edges_feature, ((0, npad), (0, 0))).reshape(EDGE_PAD // 4, 64)

    h = _node_mlp(nodes_feature, p['in_W1'].T, p['in_b1'][None],
                  p['in_W2'].T, p['in_b2'][None], D_STATE)

    h_relu = None
    for i in range(2):
        old = h
        if i > 0:
            h = h_relu
        mW1, aW1 = p['msg_W1_%d' % i], p['att_W1_%d' % i]
        eye4 = jnp.eye(4, dtype=jnp.float32)
        blk4 = lambda w: jnp.kron(eye4, w)
        w1s = blk4(jnp.concatenate([mW1[:, :32], aW1[:, :32]], 0).T)
        w1d = blk4(jnp.concatenate([mW1[:, 32:64], aW1[:, 32:64]], 0).T)
        w1e = blk4(jnp.concatenate([mW1[:, 64:], aW1[:, 64:]], 0).T)
        b1 = jnp.tile(
            jnp.concatenate([p['msg_b1_%d' % i], p['att_b1_%d' % i]]), 4)[None]
        zz = jnp.zeros((D_STATE, D_STATE), jnp.float32)
        w2m = blk4(jnp.concatenate([p['msg_W2_%d' % i].T, zz], 0))
        w2a = blk4(jnp.concatenate([zz, p['att_W2_%d' % i].T], 0))
        b2m = jnp.tile(p['msg_b2_%d' % i], 4)[None]
        b2a = jnp.tile(p['att_b2_%d' % i], 4)[None]
        wih, bih = p['gru_Wih_%d' % i].T, p['gru_bih_%d' % i][None]
        whh, bhh = p['gru_Whh_%d' % i].T, p['gru_bhh_%d' % i][None]
        for _ in range(2):
            hs, hd = _sc_gather(h, srcp, dstp)
            hs4 = hs.reshape(EDGE_PAD // 4, 128)
            hd4 = hd.reshape(EDGE_PAD // 4, 128)
            msg4 = _edge_mlp(hs4, hd4, ef4, w1s, w1d, w1e, b1, w2m, b2m,
                             w2a, b2a)
            part = _sc_scatter(msg4.reshape(EDGE_PAD, D_STATE), dst2d)
            h = _gru(part, h, wih, bih, whh, bhh)
        h, h_relu = _res(h, old, p['res_W_%d' % i].T, p['res_b_%d' % i][None])

    return _node_mlp(h, p['ro_W1'].T, p['ro_b1'][None],
                     p['ro_W2'].T, p['ro_b2'][None], 64)


# speedup vs baseline: 8.0745x; 1.0776x over previous
import functools

import jax
import jax.numpy as jnp
from jax import lax
from jax.experimental import pallas as pl
from jax.experimental.pallas import tpu as pltpu
from jax.experimental.pallas import tpu_sc as plsc

N_NODES = 10000
D_STATE = 32
NC, NS = 2, 16
NW = NC * NS
E_BLK = 128
EDGE_PAD = 327680
PER_W = EDGE_PAD // NW
NBLK = PER_W // E_BLK
N_PAD = 10016
STRIPE = N_PAD // NS

_MESH = dict(core_axis_name="c", subcore_axis_name="s")
_GK = 4


def _sc_gather(h_tbl, src_idx, dst_idx):
    out = (jax.ShapeDtypeStruct((EDGE_PAD, D_STATE), jnp.float32),
           jax.ShapeDtypeStruct((EDGE_PAD, D_STATE), jnp.float32))

    @functools.partial(
        pl.kernel, mesh=plsc.VectorSubcoreMesh(**_MESH), out_type=out,
        compiler_params=pltpu.CompilerParams(use_tc_tiling_on_sc=False),
        scratch_types=[
            pltpu.VMEM_SHARED((N_NODES, D_STATE), jnp.float32),
            pltpu.VMEM((PER_W,), jnp.int32),
            pltpu.VMEM((PER_W,), jnp.int32),
            pltpu.VMEM((_GK, E_BLK, D_STATE), jnp.float32),
            pltpu.VMEM((_GK, E_BLK, D_STATE), jnp.float32),
            pltpu.SemaphoreType.DMA,
            pltpu.SemaphoreType.DMA,
        ])
    def k(h_hbm, src_hbm, dst_hbm, hs_hbm, hd_hbm, h_sh, idx_s, idx_d, buf_s,
          buf_d, gsem, wsem):
        sid = lax.axis_index("s")
        wid = sid * NC + lax.axis_index("c")
        base = wid * PER_W
        pltpu.sync_copy(h_hbm.at[pl.ds(sid * (N_NODES // NS), N_NODES // NS)],
                        h_sh.at[pl.ds(sid * (N_NODES // NS), N_NODES // NS)])
        pltpu.sync_copy(src_hbm.at[pl.ds(base, PER_W)], idx_s)
        pltpu.sync_copy(dst_hbm.at[pl.ds(base, PER_W)], idx_d)
        plsc.subcore_barrier()

        @pl.loop(0, NBLK, step=_GK)
        def _(j0):
            s0 = j0 * E_BLK
            gathers = []
            for b in range(_GK):
                s = s0 + b * E_BLK
                gathers.append(pltpu.async_copy(
                    h_sh.at[idx_s.at[pl.ds(s, E_BLK)]], buf_s.at[b], gsem))
                gathers.append(pltpu.async_copy(
                    h_sh.at[idx_d.at[pl.ds(s, E_BLK)]], buf_d.at[b], gsem))
            writes = []
            for b in range(_GK):
                s = s0 + b * E_BLK
                gathers[2 * b].wait()
                writes.append(pltpu.async_copy(
                    buf_s.at[b], hs_hbm.at[pl.ds(base + s, E_BLK)], wsem))
                gathers[2 * b + 1].wait()
                writes.append(pltpu.async_copy(
                    buf_d.at[b], hd_hbm.at[pl.ds(base + s, E_BLK)], wsem))
            for w in writes:
                w.wait()

    return k(h_tbl, src_idx, dst_idx)


def _sc_scatter(msg, dst2d):
    @functools.partial(
        pl.kernel, mesh=plsc.VectorSubcoreMesh(**_MESH),
        out_type=jax.ShapeDtypeStruct((NC, N_PAD, D_STATE), jnp.float32),
        compiler_params=pltpu.CompilerParams(use_tc_tiling_on_sc=False),
        scratch_types=[
            pltpu.VMEM_SHARED((N_PAD, D_STATE), jnp.float32),
            pltpu.VMEM((NBLK, E_BLK), jnp.int32),
            pltpu.VMEM((_GK, E_BLK, D_STATE), jnp.float32),
            pltpu.VMEM((STRIPE, D_STATE), jnp.float32),
            pltpu.SemaphoreType.DMA,
        ])
    def k(msg_hbm, dst_hbm, out_hbm, acc, idx, mbuf, zbuf, lsem):
        cid = lax.axis_index("c")
        sid = lax.axis_index("s")
        wid = sid * NC + cid
        z = jnp.zeros((16,), jnp.float32)

        @pl.loop(0, STRIPE)
        def _(r):
            zbuf[r, pl.ds(0, 16)] = z
            zbuf[r, pl.ds(16, 16)] = z

        pltpu.sync_copy(zbuf, acc.at[pl.ds(sid * STRIPE, STRIPE)])
        plsc.subcore_barrier()

        pltpu.sync_copy(dst_hbm.at[pl.ds(wid * NBLK, NBLK)], idx)

        @pl.loop(0, NBLK, step=_GK)
        def _(j0):
            loads = []
            for b in range(_GK):
                loads.append(pltpu.async_copy(
                    msg_hbm.at[pl.ds(wid * PER_W + (j0 + b) * E_BLK, E_BLK)],
                    mbuf.at[b], lsem))
            for b in range(_GK):
                loads[b].wait()
                pltpu.sync_copy(mbuf.at[b], acc.at[idx.at[j0 + b]], add=True)

        plsc.subcore_barrier()
        pltpu.sync_copy(acc.at[pl.ds(sid * STRIPE, STRIPE)],
                        out_hbm.at[cid].at[pl.ds(sid * STRIPE, STRIPE)])

    return k(msg, dst2d)


def _dot(a, b):
    return jnp.dot(a, b, preferred_element_type=jnp.float32)


def _node_mlp_body(x_ref, w1_ref, b1_ref, w2_ref, b2_ref, o_ref):
    hid = jnp.maximum(_dot(x_ref[...], w1_ref[...]) + b1_ref[...], 0.0)
    o_ref[...] = _dot(hid, w2_ref[...]) + b2_ref[...]


def _node_mlp(x, w1, b1, w2, b2, d_out):
    return pl.pallas_call(
        _node_mlp_body,
        out_shape=jax.ShapeDtypeStruct((x.shape[0], d_out), jnp.float32),
    )(x, w1, b1, w2, b2)


def _edge_body(hs_ref, hd_ref, ef_ref, w1s_ref, w1d_ref, w1e_ref, b1_ref,
               w2m_ref, b2m_ref, w2a_ref, b2a_ref, o_ref):
    bf = jnp.bfloat16
    u = (_dot(hs_ref[...].astype(bf), w1s_ref[...])
         + _dot(hd_ref[...].astype(bf), w1d_ref[...])
         + _dot(ef_ref[...].astype(bf), w1e_ref[...]) + b1_ref[...])
    u = jnp.maximum(u, 0.0).astype(bf)
    m = _dot(u, w2m_ref[...]) + b2m_ref[...]
    a = jax.nn.sigmoid(_dot(u, w2a_ref[...]) + b2a_ref[...])
    o_ref[...] = m * a


_EB = 2048


def _edge_mlp(hs, hd, ef4, w1s, w1d, w1e, b1, w2m, b2m, w2a, b2a):
    full = lambda shape: pl.BlockSpec(shape, lambda i: (0, 0))
    ep4 = EDGE_PAD // 4
    return pl.pallas_call(
        _edge_body,
        grid=(ep4 // _EB,),
        in_specs=[
            pl.BlockSpec((_EB, 128), lambda i: (i, 0)),
            pl.BlockSpec((_EB, 128), lambda i: (i, 0)),
            pl.BlockSpec((_EB, 64), lambda i: (i, 0)),
            full((128, 256)), full((128, 256)), full((64, 256)),
            full((1, 256)), full((256, 128)), full((1, 128)),
            full((256, 128)), full((1, 128)),
        ],
        out_specs=pl.BlockSpec((_EB, 128), lambda i: (i, 0)),
        out_shape=jax.ShapeDtypeStruct((ep4, 128), jnp.float32),
        compiler_params=pltpu.CompilerParams(
            dimension_semantics=("parallel",)),
    )(hs, hd, ef4, w1s, w1d, w1e, b1, w2m, b2m, w2a, b2a)


def _gru_body(p_ref, h_ref, wih_ref, bih_ref, whh_ref, bhh_ref, o_ref):
    ms = (p_ref[0] + p_ref[1])[:N_NODES]
    h = h_ref[...]
    gi = _dot(ms, wih_ref[...]) + bih_ref[...]
    gh = _dot(h, whh_ref[...]) + bhh_ref[...]
    r = jax.nn.sigmoid(gi[:, :D_STATE] + gh[:, :D_STATE])
    z = jax.nn.sigmoid(gi[:, D_STATE:2 * D_STATE] + gh[:, D_STATE:2 * D_STATE])
    n = jnp.tanh(gi[:, 2 * D_STATE:] + r * gh[:, 2 * D_STATE:])
    o_ref[...] = (1.0 - z) * n + z * h


def _gru(part, h, wih, bih, whh, bhh):
    return pl.pallas_call(
        _gru_body,
        out_shape=jax.ShapeDtypeStruct((N_NODES, D_STATE), jnp.float32),
    )(part, h, wih, bih, whh, bhh)


def _res_body(h_ref, old_ref, w_ref, b_ref, o_ref, orelu_ref):
    x = (_dot(h_ref[...], w_ref[:D_STATE]) + _dot(old_ref[...], w_ref[D_STATE:])
         + b_ref[...])
    o_ref[...] = x
    orelu_ref[...] = jnp.maximum(x, 0.0)


def _res(h, old, w, b):
    return pl.pallas_call(
        _res_body,
        out_shape=(jax.ShapeDtypeStruct((N_NODES, D_STATE), jnp.float32),
                   jax.ShapeDtypeStruct((N_NODES, D_STATE), jnp.float32)),
    )(h, old, w, b)


def kernel(nodes_feature, edges, edges_feature, params):
    p = params
    n_edges = edges.shape[0]
    npad = EDGE_PAD - n_edges
    src = edges[:, 0].astype(jnp.int32)
    dst = edges[:, 1].astype(jnp.int32)
    srcp = jnp.pad(src, (0, npad))
    dstp = jnp.pad(dst, (0, npad))
    dst_sc = jnp.pad(dst, (0, npad), constant_values=N_NODES)
    dst2d = dst_sc.reshape(EDGE_PAD // E_BLK, E_BLK)
    ef4 = jnp.pad(edges_feature.reshape(n_edges // 4, 64),
                  ((0, npad // 4), (0, 0)))

    h = _node_mlp(nodes_feature, p['in_W1'].T, p['in_b1'][None],
                  p['in_W2'].T, p['in_b2'][None], D_STATE)

    h_relu = None
    for i in range(2):
        old = h
        if i > 0:
            h = h_relu
        mW1, aW1 = p['msg_W1_%d' % i], p['att_W1_%d' % i]
        eye4 = jnp.eye(4, dtype=jnp.float32)
        blk4 = lambda w: jnp.kron(eye4, w).astype(jnp.bfloat16)
        w1s = blk4(jnp.concatenate([mW1[:, :32], aW1[:, :32]], 0).T)
        w1d = blk4(jnp.concatenate([mW1[:, 32:64], aW1[:, 32:64]], 0).T)
        w1e = blk4(jnp.concatenate([mW1[:, 64:], aW1[:, 64:]], 0).T)
        b1 = jnp.tile(
            jnp.concatenate([p['msg_b1_%d' % i], p['att_b1_%d' % i]]), 4)[None]
        zz = jnp.zeros((D_STATE, D_STATE), jnp.float32)
        w2m = blk4(jnp.concatenate([p['msg_W2_%d' % i].T, zz], 0))
        w2a = blk4(jnp.concatenate([zz, p['att_W2_%d' % i].T], 0))
        b2m = jnp.tile(p['msg_b2_%d' % i], 4)[None]
        b2a = jnp.tile(p['att_b2_%d' % i], 4)[None]
        wih, bih = p['gru_Wih_%d' % i].T, p['gru_bih_%d' % i][None]
        whh, bhh = p['gru_Whh_%d' % i].T, p['gru_bhh_%d' % i][None]
        for _ in range(2):
            hs, hd = _sc_gather(h, srcp, dstp)
            hs4 = hs.reshape(EDGE_PAD // 4, 128)
            hd4 = hd.reshape(EDGE_PAD // 4, 128)
            msg4 = _edge_mlp(hs4, hd4, ef4, w1s, w1d, w1e, b1, w2m, b2m,
                             w2a, b2a)
            part = _sc_scatter(msg4.reshape(EDGE_PAD, D_STATE), dst2d)
            h = _gru(part, h, wih, bih, whh, bhh)
        h, h_relu = _res(h, old, p['res_W_%d' % i].T, p['res_b_%d' % i][None])

    return _node_mlp(h, p['ro_W1'].T, p['ro_b1'][None],
                     p['ro_W2'].T, p['ro_b2'][None], 64)
